# Initial kernel scaffold; baseline (speedup 1.0000x reference)
import jax
import jax.numpy as jnp
from jax.experimental import pallas as pl


def kernel(features, targets, centers):
    raise NotImplementedError("write your pallas kernel here")


# speedup vs baseline: 1.9092x; 1.9092x over previous
import functools

import jax
import jax.numpy as jnp
from jax import lax
from jax.experimental import pallas as pl
from jax.experimental.pallas import tpu as pltpu
from jax.experimental.pallas import tpu_sc as plsc

_N = 100000
_F = 128
_B = 4096
_ALPHA = 0.5
_NS = 16
_NC = 2
_L = 16
_CPT = _B // _NS
_CHUNK = 128
_NCH = _CPT // _CHUNK
_NV = _F // _L


def _sc_body(features, targets, sample_ids, zrows, zvec, ones, centers,
             out_ref, loss_out,
             tidx_v, sid_v, rep_v, fbuf, cbuf, abuf, cnt_v, ones_v, lidx_v,
             lbuf, rep_sh, a_sh, cnt_sh, ladd_sh):
    c = lax.axis_index("c")
    s = lax.axis_index("s")
    base = s * _CPT

    pltpu.sync_copy(zrows.at[pl.ds(base, _CPT)], a_sh.at[pl.ds(base, _CPT)])
    pltpu.sync_copy(zvec.at[pl.ds(base, _CPT)], cnt_sh.at[pl.ds(base, _CPT)])
    pltpu.sync_copy(ones, ones_v)
    pltpu.sync_copy(sample_ids.at[pl.ds(0, _L)], lidx_v)

    @pl.when(s == 0)
    def _():
        pltpu.sync_copy(zvec.at[pl.ds(0, _CHUNK)], ladd_sh)

    for k in range(_NCH):
        off = base + k * _CHUNK
        pltpu.sync_copy(targets.at[pl.ds(off, _CHUNK)], tidx_v.at[k])
        pltpu.sync_copy(sample_ids.at[pl.ds(off, _CHUNK)], sid_v.at[k])
    for k in range(_NCH):
        pltpu.sync_copy(sid_v.at[k], rep_sh.at[tidx_v.at[k]])
    plsc.subcore_barrier()

    lsum = jnp.zeros((_L,), jnp.float32)
    for k in range(_NCH):
        off = base + k * _CHUNK
        pltpu.sync_copy(rep_sh.at[tidx_v.at[k]], rep_v.at[k])
        pltpu.sync_copy(centers.at[tidx_v.at[k]], cbuf.at[k])
        pltpu.sync_copy(features.at[pl.ds(off, _CHUNK)], fbuf)

        def dbody(r, acc, k=k):
            for j in range(_NV):
                cv = cbuf[k, r, pl.ds(j * _L, _L)]
                fv = fbuf[r, pl.ds(j * _L, _L)]
                d = cv - fv
                fbuf[r, pl.ds(j * _L, _L)] = d
                acc = acc + d * d
            return acc

        lsum = pl.loop(0, _CHUNK, init_carry=lsum)(dbody)
        pltpu.sync_copy(fbuf, a_sh.at[rep_v.at[k]], add=True)
        pltpu.sync_copy(ones_v, cnt_sh.at[rep_v.at[k]], add=True)

    lbuf[...] = lsum
    pltpu.sync_copy(lbuf, ladd_sh.at[lidx_v], add=True)
    plsc.subcore_barrier()

    for k in range(_NCH):
        @pl.when(c == (k % _NC))
        def _(k=k):
            pltpu.sync_copy(a_sh.at[rep_v.at[k]], abuf)
            pltpu.sync_copy(cnt_sh.at[rep_v.at[k]], cnt_v)

            @pl.loop(0, _CHUNK // _L)
            def _(g, k=k):
                nv = cnt_v[pl.ds(g * _L, _L)]
                sv = _ALPHA / (nv + 1.0)
                for i in range(_L):
                    r = g * _L + i
                    scale = sv[i]
                    for j in range(_NV):
                        av = abuf[r, pl.ds(j * _L, _L)]
                        cv = cbuf[k, r, pl.ds(j * _L, _L)]
                        abuf[r, pl.ds(j * _L, _L)] = cv - av * scale

            pltpu.sync_copy(abuf, out_ref.at[tidx_v.at[k]])

    @pl.when((c == 0) & (s == 0))
    def _():
        pltpu.sync_copy(ladd_sh, cnt_v)
        acc = cnt_v[pl.ds(0, _L)]
        total = acc[0]
        for i in range(1, _L):
            total = total + acc[i]
        total = total * (1.0 / (_B * _F))
        lbuf[...] = jnp.full((_L,), 0.0, jnp.float32) + total
        pltpu.sync_copy(lbuf, loss_out)


@functools.lru_cache(maxsize=1)
def _make_sc_call():
    mesh = plsc.VectorSubcoreMesh(core_axis_name="c", subcore_axis_name="s",
                                  num_cores=_NC, num_subcores=_NS)
    return pl.kernel(
        _sc_body,
        out_type=jax.ShapeDtypeStruct((_L,), jnp.float32),
        mesh=mesh,
        scratch_types=_SCRATCH,
    )

_SCRATCH = [
        pltpu.VMEM((_NCH, _CHUNK), jnp.int32),
        pltpu.VMEM((_NCH, _CHUNK), jnp.int32),
        pltpu.VMEM((_NCH, _CHUNK), jnp.int32),
        pltpu.VMEM((_CHUNK, _F), jnp.float32),
        pltpu.VMEM((_NCH, _CHUNK, _F), jnp.float32),
        pltpu.VMEM((_CHUNK, _F), jnp.float32),
        pltpu.VMEM((_CHUNK,), jnp.float32),
        pltpu.VMEM((_CHUNK,), jnp.float32),
        pltpu.VMEM((_L,), jnp.int32),
        pltpu.VMEM((_L,), jnp.float32),
        pltpu.VMEM_SHARED((_N,), jnp.int32),
        pltpu.VMEM_SHARED((_B, _F), jnp.float32),
        pltpu.VMEM_SHARED((_B,), jnp.float32),
        pltpu.VMEM_SHARED((_CHUNK,), jnp.float32),
]

def kernel(features, targets, centers):
    _sc_call = _make_sc_call()
    sample_ids = jnp.arange(_B, dtype=jnp.int32)
    zrows = jnp.zeros((_B, _F), jnp.float32)
    zvec = jnp.zeros((_B,), jnp.float32)
    ones = jnp.ones((_CHUNK,), jnp.float32)
    out_ref = jax.new_ref(centers)
    loss_vec = _sc_call(features, targets, sample_ids, zrows, zvec, ones,
                        centers, out_ref)
    return jnp.reshape(loss_vec[0], ()), out_ref[...]
